# Initial kernel scaffold; baseline (speedup 1.0000x reference)
import functools

import jax
import jax.numpy as jnp
from jax import lax
from jax.experimental import pallas as pl
from jax.experimental.pallas import tpu as pltpu
from jax.experimental.pallas import tpu_sc as plsc

N = 10000
E = 320000
D = 128
H1, C1 = 8, 16
F1 = H1 * C1
C2 = 40
C2P = 48
ACC1_W = F1 + 16
ACC2_W = 64

NC, NS = 2, 16
NW = NC * NS
CHUNK = 80
BN = 500
BE = 500


def _lrelu(v):
    return jnp.where(v >= 0, v, 0.2 * v)


def _make_gather2(width, n_edges):
    per_w = n_edges // NW
    n_chunks = per_w // CHUNK
    mesh = plsc.VectorSubcoreMesh(core_axis_name="c", subcore_axis_name="s")

    @functools.partial(
        pl.kernel,
        out_type=(jax.ShapeDtypeStruct((n_edges, width), jnp.float32),
                  jax.ShapeDtypeStruct((n_edges, width), jnp.float32)),
        mesh=mesh,
        scratch_types=[
            pltpu.VMEM((CHUNK,), jnp.int32),
            pltpu.VMEM((CHUNK,), jnp.int32),
            pltpu.VMEM((CHUNK, width), jnp.float32),
            pltpu.VMEM((CHUNK, width), jnp.float32),
            pltpu.SemaphoreType.DMA,
            pltpu.SemaphoreType.DMA,
        ],
    )
    def gather_kernel(tl, tr, src, dst, xj_out, xi_out,
                      sidx, didx, xj_v, xi_v, sem1, sem2):
        c = lax.axis_index("c")
        s = lax.axis_index("s")
        base = (c * NS + s) * per_w

        def body(i, carry):
            b = base + i * CHUNK
            pltpu.sync_copy(src.at[pl.ds(b, CHUNK)], sidx)
            pltpu.sync_copy(dst.at[pl.ds(b, CHUNK)], didx)
            cp1 = pltpu.async_copy(tl.at[sidx], xj_v, sem1)
            cp2 = pltpu.async_copy(tr.at[didx], xi_v, sem2)
            cp1.wait()
            cp2.wait()
            pltpu.sync_copy(xj_v, xj_out.at[pl.ds(b, CHUNK)])
            pltpu.sync_copy(xi_v, xi_out.at[pl.ds(b, CHUNK)])
            return carry

        lax.fori_loop(0, n_chunks, body, 0)

    return gather_kernel


def _make_scatter(width, n_edges, n_nodes):
    per_w = n_edges // NW
    n_chunks = per_w // CHUNK
    rows_per_tile = n_nodes // NS
    mesh = plsc.VectorSubcoreMesh(core_axis_name="c", subcore_axis_name="s")

    @functools.partial(
        pl.kernel,
        out_type=jax.ShapeDtypeStruct((2 * n_nodes, width), jnp.float32),
        mesh=mesh,
        scratch_types=[
            pltpu.VMEM((CHUNK,), jnp.int32),
            pltpu.VMEM((CHUNK, width), jnp.float32),
            pltpu.VMEM_SHARED((n_nodes, width), jnp.float32),
        ],
    )
    def scatter_kernel(vals, dstidx, zeros, out, idx_v, val_v, acc):
        c = lax.axis_index("c")
        s = lax.axis_index("s")
        r0 = s * rows_per_tile
        pltpu.sync_copy(zeros, acc.at[pl.ds(r0, rows_per_tile)])
        plsc.subcore_barrier()
        base = (c * NS + s) * per_w

        def body(i, carry):
            b = base + i * CHUNK
            pltpu.sync_copy(dstidx.at[pl.ds(b, CHUNK)], idx_v)
            pltpu.sync_copy(vals.at[pl.ds(b, CHUNK)], val_v)
            pltpu.sync_copy(val_v, acc.at[idx_v], add=True)
            return carry

        lax.fori_loop(0, n_chunks, body, 0)
        plsc.subcore_barrier()
        pltpu.sync_copy(acc.at[pl.ds(r0, rows_per_tile)],
                        out.at[pl.ds(c * n_nodes + r0, rows_per_tile)])

    return scatter_kernel


def _project(x, wl, bl, wr, br):
    n, d = x.shape
    f = wl.shape[1]

    def body(x_ref, wl_ref, bl_ref, wr_ref, br_ref, xl_ref, xr_ref):
        xb = x_ref[...]
        xl_ref[...] = jnp.dot(xb, wl_ref[...],
                              preferred_element_type=jnp.float32) + bl_ref[...]
        xr_ref[...] = jnp.dot(xb, wr_ref[...],
                              preferred_element_type=jnp.float32) + br_ref[...]

    return pl.pallas_call(
        body,
        grid=(n // BN,),
        in_specs=[
            pl.BlockSpec((BN, d), lambda i: (i, 0)),
            pl.BlockSpec((d, f), lambda i: (0, 0)),
            pl.BlockSpec((1, f), lambda i: (0, 0)),
            pl.BlockSpec((d, f), lambda i: (0, 0)),
            pl.BlockSpec((1, f), lambda i: (0, 0)),
        ],
        out_specs=[pl.BlockSpec((BN, f), lambda i: (i, 0))] * 2,
        out_shape=[jax.ShapeDtypeStruct((n, f), jnp.float32)] * 2,
    )(x, wl, bl.reshape(1, -1), wr, br.reshape(1, -1))


def _edge_compute1(xi, xj, a_mat, b_mat):
    def body(xi_ref, xj_ref, a_ref, b_ref, v_ref):
        xi_ = xi_ref[...]
        xj_ = xj_ref[...]
        l = _lrelu(xi_ + xj_)
        w = jnp.exp(jnp.dot(l, a_ref[...],
                            preferred_element_type=jnp.float32))
        wide = jnp.dot(w, b_ref[...],
                       preferred_element_type=jnp.float32)
        v_ref[...] = jnp.concatenate([wide * xj_, w, jnp.zeros_like(w)],
                                     axis=1)

    return pl.pallas_call(
        body,
        grid=(E // BE,),
        in_specs=[
            pl.BlockSpec((BE, F1), lambda i: (i, 0)),
            pl.BlockSpec((BE, F1), lambda i: (i, 0)),
            pl.BlockSpec((F1, H1), lambda i: (0, 0)),
            pl.BlockSpec((H1, F1), lambda i: (0, 0)),
        ],
        out_specs=pl.BlockSpec((BE, ACC1_W), lambda i: (i, 0)),
        out_shape=jax.ShapeDtypeStruct((E, ACC1_W), jnp.float32),
    )(xi, xj, a_mat, b_mat)


def _edge_compute2(xi, xj, att2p):
    def body(xi_ref, xj_ref, a_ref, v_ref):
        xi_ = xi_ref[...]
        xj_ = xj_ref[...]
        l = _lrelu(xi_ + xj_)
        w = jnp.exp(jnp.dot(l, a_ref[...],
                            preferred_element_type=jnp.float32))
        z = jnp.zeros((xi_.shape[0], ACC2_W - C2P - 1), jnp.float32)
        v_ref[...] = jnp.concatenate([w * xj_, w, z], axis=1)

    return pl.pallas_call(
        body,
        grid=(E // BE,),
        in_specs=[
            pl.BlockSpec((BE, C2P), lambda i: (i, 0)),
            pl.BlockSpec((BE, C2P), lambda i: (i, 0)),
            pl.BlockSpec((C2P, 1), lambda i: (0, 0)),
        ],
        out_specs=pl.BlockSpec((BE, ACC2_W), lambda i: (i, 0)),
        out_shape=jax.ShapeDtypeStruct((E, ACC2_W), jnp.float32),
    )(xi, xj, att2p)


def _mid_stage(acc0, acc1, xl1, xr1, a_mat, b_mat, bias1,
               wl2p, bl2p, wr2p, br2p, att2p):
    def body(a0_ref, a1_ref, xl_ref, xr_ref, a_ref, b_ref, b1_ref,
             wl2_ref, bl2_ref, wr2_ref, br2_ref, att2_ref,
             xl2_ref, xr2_ref, s2_ref):
        xl_ = xl_ref[...]
        xr_ = xr_ref[...]
        l = _lrelu(xl_ + xr_)
        wii = jnp.exp(jnp.dot(l, a_ref[...],
                              preferred_element_type=jnp.float32))
        t0 = a0_ref[...]
        t1 = a1_ref[...]
        num = (t0[:, :F1] + t1[:, :F1]
               + jnp.dot(wii, b_ref[...],
                         preferred_element_type=jnp.float32) * xl_)
        den = t0[:, F1:F1 + H1] + t1[:, F1:F1 + H1] + wii
        inv = 1.0 / (den + 1e-16)
        hin = num * jnp.dot(inv, b_ref[...],
                            preferred_element_type=jnp.float32) + b1_ref[...]
        h = jnp.where(hin > 0, hin, jnp.expm1(hin))
        xl2 = jnp.dot(h, wl2_ref[...],
                      preferred_element_type=jnp.float32) + bl2_ref[...]
        xr2 = jnp.dot(h, wr2_ref[...],
                      preferred_element_type=jnp.float32) + br2_ref[...]
        l2 = _lrelu(xl2 + xr2)
        w2 = jnp.exp(jnp.dot(l2, att2_ref[...],
                             preferred_element_type=jnp.float32))
        z = jnp.zeros((xl2.shape[0], ACC2_W - C2P - 1), jnp.float32)
        xl2_ref[...] = xl2
        xr2_ref[...] = xr2
        s2_ref[...] = jnp.concatenate([w2 * xl2, w2, z], axis=1)

    return pl.pallas_call(
        body,
        grid=(N // BN,),
        in_specs=[
            pl.BlockSpec((BN, ACC1_W), lambda i: (i, 0)),
            pl.BlockSpec((BN, ACC1_W), lambda i: (i, 0)),
            pl.BlockSpec((BN, F1), lambda i: (i, 0)),
            pl.BlockSpec((BN, F1), lambda i: (i, 0)),
            pl.BlockSpec((F1, H1), lambda i: (0, 0)),
            pl.BlockSpec((H1, F1), lambda i: (0, 0)),
            pl.BlockSpec((1, F1), lambda i: (0, 0)),
            pl.BlockSpec((F1, C2P), lambda i: (0, 0)),
            pl.BlockSpec((1, C2P), lambda i: (0, 0)),
            pl.BlockSpec((F1, C2P), lambda i: (0, 0)),
            pl.BlockSpec((1, C2P), lambda i: (0, 0)),
            pl.BlockSpec((C2P, 1), lambda i: (0, 0)),
        ],
        out_specs=[
            pl.BlockSpec((BN, C2P), lambda i: (i, 0)),
            pl.BlockSpec((BN, C2P), lambda i: (i, 0)),
            pl.BlockSpec((BN, ACC2_W), lambda i: (i, 0)),
        ],
        out_shape=[
            jax.ShapeDtypeStruct((N, C2P), jnp.float32),
            jax.ShapeDtypeStruct((N, C2P), jnp.float32),
            jax.ShapeDtypeStruct((N, ACC2_W), jnp.float32),
        ],
    )(acc0, acc1, xl1, xr1, a_mat, b_mat, bias1,
      wl2p, bl2p, wr2p, br2p, att2p)


def _final_stage(acc0, acc1, s2, bias2):
    def body(a0_ref, a1_ref, s2_ref, b2_ref, out_ref):
        t = a0_ref[...] + a1_ref[...] + s2_ref[...]
        den = t[:, C2P:C2P + 1]
        out_ref[...] = t[:, :C2] / (den + 1e-16) + b2_ref[...]

    return pl.pallas_call(
        body,
        grid=(N // BN,),
        in_specs=[
            pl.BlockSpec((BN, ACC2_W), lambda i: (i, 0)),
            pl.BlockSpec((BN, ACC2_W), lambda i: (i, 0)),
            pl.BlockSpec((BN, ACC2_W), lambda i: (i, 0)),
            pl.BlockSpec((1, C2), lambda i: (0, 0)),
        ],
        out_specs=pl.BlockSpec((BN, C2), lambda i: (i, 0)),
        out_shape=jax.ShapeDtypeStruct((N, C2), jnp.float32),
    )(acc0, acc1, s2, bias2.reshape(1, -1))


_gather128 = _make_gather2(F1, E)
_gather48 = _make_gather2(C2P, E)
_scatter144 = _make_scatter(ACC1_W, E, N)
_scatter64 = _make_scatter(ACC2_W, E, N)


def kernel(x, edge_index, Wl1, bl1, Wr1, br1, att1, bias1,
           Wl2, bl2, Wr2, br2, att2, bias2):
    src = edge_index[0]
    dst = edge_index[1]

    a1 = (att1[:, :, None] * jnp.eye(H1, dtype=jnp.float32)[:, None, :]
          ).reshape(F1, H1)
    bm = jnp.repeat(jnp.eye(H1, dtype=jnp.float32), C1, axis=1)
    att2p = jnp.zeros((C2P, 1), jnp.float32).at[:C2, 0].set(att2[0])
    wl2p = jnp.pad(Wl2, ((0, 0), (0, C2P - C2)))
    wr2p = jnp.pad(Wr2, ((0, 0), (0, C2P - C2)))
    bl2p = jnp.pad(bl2, (0, C2P - C2)).reshape(1, -1)
    br2p = jnp.pad(br2, (0, C2P - C2)).reshape(1, -1)
    zeros1 = jnp.zeros((N // NS, ACC1_W), jnp.float32)
    zeros2 = jnp.zeros((N // NS, ACC2_W), jnp.float32)

    xl1, xr1 = _project(x, Wl1, bl1, Wr1, br1)
    xj1, xi1 = _gather128(xl1, xr1, src, dst)
    v1 = _edge_compute1(xi1, xj1, a1, bm)
    acc1 = _scatter144(v1, dst, zeros1)
    xl2, xr2, s2 = _mid_stage(acc1[:N], acc1[N:], xl1, xr1, a1, bm,
                              bias1.reshape(1, -1), wl2p, bl2p, wr2p, br2p,
                              att2p)
    xj2, xi2 = _gather48(xl2, xr2, src, dst)
    v2 = _edge_compute2(xi2, xj2, att2p)
    acc2 = _scatter64(v2, dst, zeros2)
    return _final_stage(acc2[:N], acc2[N:], s2, bias2)


# speedup vs baseline: 19.5897x; 19.5897x over previous
import functools

import jax
import jax.numpy as jnp
from jax import lax
from jax.experimental import pallas as pl
from jax.experimental.pallas import tpu as pltpu
from jax.experimental.pallas import tpu_sc as plsc

N = 10000
E = 320000
D = 128
H1, C1 = 8, 16
F1 = H1 * C1
C2 = 40
C2P = 48
ACC1_W = F1 + 16
ACC2_W = 64

NC, NS = 2, 16
NW = NC * NS
CHUNK = 80
BN = 1000
BE = 1000


def _lrelu(v):
    return jnp.where(v >= 0, v, 0.2 * v)


_SC_PARAMS = pltpu.CompilerParams(use_tc_tiling_on_sc=False)


def _make_gather2(width, n_edges):
    per_w = n_edges // NW
    n_chunks = per_w // CHUNK
    mesh = plsc.VectorSubcoreMesh(core_axis_name="c", subcore_axis_name="s")

    @functools.partial(
        pl.kernel,
        out_type=(jax.ShapeDtypeStruct((n_edges, width), jnp.float32),
                  jax.ShapeDtypeStruct((n_edges, width), jnp.float32)),
        mesh=mesh,
        scratch_types=[
            pltpu.VMEM((CHUNK,), jnp.int32),
            pltpu.VMEM((CHUNK,), jnp.int32),
            pltpu.VMEM((CHUNK, width), jnp.float32),
            pltpu.VMEM((CHUNK, width), jnp.float32),
            pltpu.SemaphoreType.DMA,
            pltpu.SemaphoreType.DMA,
        ],
        compiler_params=_SC_PARAMS,
    )
    def gather_kernel(tl, tr, src, dst, xj_out, xi_out,
                      sidx, didx, xj_v, xi_v, sem1, sem2):
        c = lax.axis_index("c")
        s = lax.axis_index("s")
        base = (c * NS + s) * per_w

        def body(i, carry):
            b = base + i * CHUNK
            pltpu.sync_copy(src.at[pl.ds(b, CHUNK)], sidx)
            pltpu.sync_copy(dst.at[pl.ds(b, CHUNK)], didx)
            cp1 = pltpu.async_copy(tl.at[sidx], xj_v, sem1)
            cp2 = pltpu.async_copy(tr.at[didx], xi_v, sem2)
            cp1.wait()
            cp2.wait()
            pltpu.sync_copy(xj_v, xj_out.at[pl.ds(b, CHUNK)])
            pltpu.sync_copy(xi_v, xi_out.at[pl.ds(b, CHUNK)])
            return carry

        lax.fori_loop(0, n_chunks, body, 0)

    return gather_kernel


NP = 10240


def _make_scatter(width, n_edges, n_nodes):
    del n_nodes
    per_w = n_edges // NW
    n_chunks = per_w // CHUNK
    rows_per_tile = NP // NS
    mesh = plsc.VectorSubcoreMesh(core_axis_name="c", subcore_axis_name="s")

    @functools.partial(
        pl.kernel,
        out_type=jax.ShapeDtypeStruct((2 * NP, width), jnp.float32),
        mesh=mesh,
        scratch_types=[
            pltpu.VMEM((CHUNK,), jnp.int32),
            pltpu.VMEM((CHUNK, width), jnp.float32),
            pltpu.VMEM_SHARED((NP, width), jnp.float32),
        ],
        compiler_params=_SC_PARAMS,
    )
    def scatter_kernel(vals, dstidx, zeros, out, idx_v, val_v, acc):
        c = lax.axis_index("c")
        s = lax.axis_index("s")
        r0 = s * rows_per_tile
        pltpu.sync_copy(zeros, acc.at[pl.ds(r0, rows_per_tile)])
        plsc.subcore_barrier()
        base = (c * NS + s) * per_w

        def body(i, carry):
            b = base + i * CHUNK
            pltpu.sync_copy(dstidx.at[pl.ds(b, CHUNK)], idx_v)
            pltpu.sync_copy(vals.at[pl.ds(b, CHUNK)], val_v)
            pltpu.sync_copy(val_v, acc.at[idx_v], add=True)
            return carry

        lax.fori_loop(0, n_chunks, body, 0)
        plsc.subcore_barrier()
        pltpu.sync_copy(acc.at[pl.ds(r0, rows_per_tile)],
                        out.at[pl.ds(c * NP + r0, rows_per_tile)])

    return scatter_kernel


def _project(x, wl, bl, wr, br):
    n, d = x.shape
    f = wl.shape[1]

    def body(x_ref, wl_ref, bl_ref, wr_ref, br_ref, xl_ref, xr_ref):
        xb = x_ref[...]
        xl_ref[...] = jnp.dot(xb, wl_ref[...],
                              preferred_element_type=jnp.float32) + bl_ref[...]
        xr_ref[...] = jnp.dot(xb, wr_ref[...],
                              preferred_element_type=jnp.float32) + br_ref[...]

    return pl.pallas_call(
        body,
        grid=(n // BN,),
        in_specs=[
            pl.BlockSpec((BN, d), lambda i: (i, 0)),
            pl.BlockSpec((d, f), lambda i: (0, 0)),
            pl.BlockSpec((1, f), lambda i: (0, 0)),
            pl.BlockSpec((d, f), lambda i: (0, 0)),
            pl.BlockSpec((1, f), lambda i: (0, 0)),
        ],
        out_specs=[pl.BlockSpec((BN, f), lambda i: (i, 0))] * 2,
        out_shape=[jax.ShapeDtypeStruct((n, f), jnp.float32)] * 2,
    )(x, wl, bl.reshape(1, -1), wr, br.reshape(1, -1))


def _edge_compute1(xi, xj, a_mat, b_mat):
    def body(xi_ref, xj_ref, a_ref, b_ref, v_ref):
        xi_ = xi_ref[...]
        xj_ = xj_ref[...]
        l = _lrelu(xi_ + xj_)
        w = jnp.exp(jnp.dot(l, a_ref[...],
                            preferred_element_type=jnp.float32))
        wide = jnp.dot(w, b_ref[...],
                       preferred_element_type=jnp.float32)
        v_ref[...] = jnp.concatenate([wide * xj_, w, jnp.zeros_like(w)],
                                     axis=1)

    return pl.pallas_call(
        body,
        grid=(E // BE,),
        in_specs=[
            pl.BlockSpec((BE, F1), lambda i: (i, 0)),
            pl.BlockSpec((BE, F1), lambda i: (i, 0)),
            pl.BlockSpec((F1, H1), lambda i: (0, 0)),
            pl.BlockSpec((H1, F1), lambda i: (0, 0)),
        ],
        out_specs=pl.BlockSpec((BE, ACC1_W), lambda i: (i, 0)),
        out_shape=jax.ShapeDtypeStruct((E, ACC1_W), jnp.float32),
    )(xi, xj, a_mat, b_mat)


def _edge_compute2(xi, xj, att2p):
    def body(xi_ref, xj_ref, a_ref, v_ref):
        xi_ = xi_ref[...]
        xj_ = xj_ref[...]
        l = _lrelu(xi_ + xj_)
        w = jnp.exp(jnp.dot(l, a_ref[...],
                            preferred_element_type=jnp.float32))
        z = jnp.zeros((xi_.shape[0], ACC2_W - C2P - 1), jnp.float32)
        v_ref[...] = jnp.concatenate([w * xj_, w, z], axis=1)

    return pl.pallas_call(
        body,
        grid=(E // BE,),
        in_specs=[
            pl.BlockSpec((BE, C2P), lambda i: (i, 0)),
            pl.BlockSpec((BE, C2P), lambda i: (i, 0)),
            pl.BlockSpec((C2P, 1), lambda i: (0, 0)),
        ],
        out_specs=pl.BlockSpec((BE, ACC2_W), lambda i: (i, 0)),
        out_shape=jax.ShapeDtypeStruct((E, ACC2_W), jnp.float32),
    )(xi, xj, att2p)


def _mid_stage(acc0, acc1, xl1, xr1, a_mat, b_mat, bias1,
               wl2p, bl2p, wr2p, br2p, att2p):
    def body(a0_ref, a1_ref, xl_ref, xr_ref, a_ref, b_ref, b1_ref,
             wl2_ref, bl2_ref, wr2_ref, br2_ref, att2_ref,
             xl2_ref, xr2_ref, s2_ref):
        xl_ = xl_ref[...]
        xr_ = xr_ref[...]
        l = _lrelu(xl_ + xr_)
        wii = jnp.exp(jnp.dot(l, a_ref[...],
                              preferred_element_type=jnp.float32))
        t0 = a0_ref[...]
        t1 = a1_ref[...]
        num = (t0[:, :F1] + t1[:, :F1]
               + jnp.dot(wii, b_ref[...],
                         preferred_element_type=jnp.float32) * xl_)
        den = t0[:, F1:F1 + H1] + t1[:, F1:F1 + H1] + wii
        inv = 1.0 / (den + 1e-16)
        hin = num * jnp.dot(inv, b_ref[...],
                            preferred_element_type=jnp.float32) + b1_ref[...]
        h = jnp.where(hin > 0, hin, jnp.exp(jnp.minimum(hin, 0.0)) - 1.0)
        xl2 = jnp.dot(h, wl2_ref[...],
                      preferred_element_type=jnp.float32) + bl2_ref[...]
        xr2 = jnp.dot(h, wr2_ref[...],
                      preferred_element_type=jnp.float32) + br2_ref[...]
        l2 = _lrelu(xl2 + xr2)
        w2 = jnp.exp(jnp.dot(l2, att2_ref[...],
                             preferred_element_type=jnp.float32))
        z = jnp.zeros((xl2.shape[0], ACC2_W - C2P - 1), jnp.float32)
        xl2_ref[...] = xl2
        xr2_ref[...] = xr2
        s2_ref[...] = jnp.concatenate([w2 * xl2, w2, z], axis=1)

    return pl.pallas_call(
        body,
        grid=(N // BN,),
        in_specs=[
            pl.BlockSpec((BN, ACC1_W), lambda i: (i, 0)),
            pl.BlockSpec((BN, ACC1_W), lambda i: (i, 0)),
            pl.BlockSpec((BN, F1), lambda i: (i, 0)),
            pl.BlockSpec((BN, F1), lambda i: (i, 0)),
            pl.BlockSpec((F1, H1), lambda i: (0, 0)),
            pl.BlockSpec((H1, F1), lambda i: (0, 0)),
            pl.BlockSpec((1, F1), lambda i: (0, 0)),
            pl.BlockSpec((F1, C2P), lambda i: (0, 0)),
            pl.BlockSpec((1, C2P), lambda i: (0, 0)),
            pl.BlockSpec((F1, C2P), lambda i: (0, 0)),
            pl.BlockSpec((1, C2P), lambda i: (0, 0)),
            pl.BlockSpec((C2P, 1), lambda i: (0, 0)),
        ],
        out_specs=[
            pl.BlockSpec((BN, C2P), lambda i: (i, 0)),
            pl.BlockSpec((BN, C2P), lambda i: (i, 0)),
            pl.BlockSpec((BN, ACC2_W), lambda i: (i, 0)),
        ],
        out_shape=[
            jax.ShapeDtypeStruct((N, C2P), jnp.float32),
            jax.ShapeDtypeStruct((N, C2P), jnp.float32),
            jax.ShapeDtypeStruct((N, ACC2_W), jnp.float32),
        ],
    )(acc0, acc1, xl1, xr1, a_mat, b_mat, bias1,
      wl2p, bl2p, wr2p, br2p, att2p)


def _final_stage(acc0, acc1, s2, bias2):
    def body(a0_ref, a1_ref, s2_ref, b2_ref, out_ref):
        t = a0_ref[...] + a1_ref[...] + s2_ref[...]
        den = t[:, C2P:C2P + 1]
        out_ref[...] = t[:, :C2] / (den + 1e-16) + b2_ref[...]

    return pl.pallas_call(
        body,
        grid=(N // BN,),
        in_specs=[
            pl.BlockSpec((BN, ACC2_W), lambda i: (i, 0)),
            pl.BlockSpec((BN, ACC2_W), lambda i: (i, 0)),
            pl.BlockSpec((BN, ACC2_W), lambda i: (i, 0)),
            pl.BlockSpec((1, C2), lambda i: (0, 0)),
        ],
        out_specs=pl.BlockSpec((BN, C2), lambda i: (i, 0)),
        out_shape=jax.ShapeDtypeStruct((N, C2), jnp.float32),
    )(acc0, acc1, s2, bias2.reshape(1, -1))


_gather128 = _make_gather2(F1, E)
_gather48 = _make_gather2(C2P, E)
_scatter144 = _make_scatter(ACC1_W, E, N)
_scatter64 = _make_scatter(ACC2_W, E, N)


def kernel(x, edge_index, Wl1, bl1, Wr1, br1, att1, bias1,
           Wl2, bl2, Wr2, br2, att2, bias2):
    src = edge_index[0]
    dst = edge_index[1]

    a1 = (att1[:, :, None] * jnp.eye(H1, dtype=jnp.float32)[:, None, :]
          ).reshape(F1, H1)
    bm = jnp.repeat(jnp.eye(H1, dtype=jnp.float32), C1, axis=1)
    att2p = jnp.zeros((C2P, 1), jnp.float32).at[:C2, 0].set(att2[0])
    wl2p = jnp.pad(Wl2, ((0, 0), (0, C2P - C2)))
    wr2p = jnp.pad(Wr2, ((0, 0), (0, C2P - C2)))
    bl2p = jnp.pad(bl2, (0, C2P - C2)).reshape(1, -1)
    br2p = jnp.pad(br2, (0, C2P - C2)).reshape(1, -1)
    zeros1 = jnp.zeros((NP // NS, ACC1_W), jnp.float32)
    zeros2 = jnp.zeros((NP // NS, ACC2_W), jnp.float32)

    xl1, xr1 = _project(x, Wl1, bl1, Wr1, br1)
    xj1, xi1 = _gather128(xl1, xr1, src, dst)
    v1 = _edge_compute1(xi1, xj1, a1, bm)
    acc1 = _scatter144(v1, dst, zeros1)
    xl2, xr2, s2 = _mid_stage(acc1[:N], acc1[NP:NP + N], xl1, xr1, a1, bm,
                              bias1.reshape(1, -1), wl2p, bl2p, wr2p, br2p,
                              att2p)
    xj2, xi2 = _gather48(xl2, xr2, src, dst)
    v2 = _edge_compute2(xi2, xj2, att2p)
    acc2 = _scatter64(v2, dst, zeros2)
    return _final_stage(acc2[:N], acc2[NP:NP + N], s2, bias2)


# speedup vs baseline: 47.7929x; 2.4397x over previous
import functools

import jax
import jax.numpy as jnp
from jax import lax
from jax.experimental import pallas as pl
from jax.experimental.pallas import tpu as pltpu
from jax.experimental.pallas import tpu_sc as plsc

N = 10000
E = 320000
D = 128
H1, C1 = 8, 16
F1 = H1 * C1
C2 = 40
C2P = 48
ACC1_W = F1 + 16
ACC2_W = 64

NC, NS = 2, 16
NW = NC * NS
CH = 40
G = 10
NP = 10112
BN = 1000


def _lrelu(v):
    return jnp.where(v >= 0, v, 0.2 * v)


_SC_PARAMS = pltpu.CompilerParams(use_tc_tiling_on_sc=False,
                                  needs_layout_passes=False)


def _make_fused_edge_pass(width, acc_w, n_heads):
    per_w = E // NW
    n_ch = per_w // CH
    n_grp = n_ch // G
    n_vr = width // 16
    rows_per_tile = NP // NS
    mesh = plsc.VectorSubcoreMesh(core_axis_name="c", subcore_axis_name="s")

    @functools.partial(
        pl.kernel,
        out_type=jax.ShapeDtypeStruct((2 * NP, acc_w), jnp.float32),
        mesh=mesh,
        scratch_types=[
            pltpu.VMEM((3, G, CH), jnp.int32),
            pltpu.VMEM((3, G, CH), jnp.int32),
            pltpu.VMEM((2, CH, width), jnp.float32),
            pltpu.VMEM((2, CH, width), jnp.float32),
            pltpu.VMEM((2, CH, acc_w), jnp.float32),
            pltpu.VMEM((width // 16, 16), jnp.float32),
            pltpu.VMEM_SHARED((NP, acc_w), jnp.float32),
            pltpu.SemaphoreType.DMA,
            pltpu.SemaphoreType.DMA,
            pltpu.SemaphoreType.DMA,
            pltpu.SemaphoreType.DMA,
            pltpu.SemaphoreType.DMA,
            pltpu.SemaphoreType.DMA,
            pltpu.SemaphoreType.DMA,
        ],
        compiler_params=_SC_PARAMS,
    )
    def fused_kernel(tl, tr, s3d, d3d, att, zeros, out,
                     sidx, didx, xj_v, xi_v, ov, att_v, acc,
                     gj0, gj1, gi0, gi1, ss0, ss1, slab_sem):
        gj = (gj0, gj1)
        gi = (gi0, gi1)
        ss = (ss0, ss1)
        c = lax.axis_index("c")
        s = lax.axis_index("s")
        wid = c * NS + s

        pltpu.sync_copy(att, att_v)
        r0 = s * rows_per_tile
        pltpu.sync_copy(zeros, acc.at[pl.ds(r0, rows_per_tile)])

        def start_slab(g, slot):
            pltpu.async_copy(s3d.at[wid, g], sidx.at[slot], slab_sem)
            pltpu.async_copy(d3d.at[wid, g], didx.at[slot], slab_sem)

        def wait_slab():
            pltpu.make_async_copy(s3d.at[wid, 0], sidx.at[0],
                                  slab_sem).wait()
            pltpu.make_async_copy(d3d.at[wid, 0], didx.at[0],
                                  slab_sem).wait()

        start_slab(0, 0)
        wait_slab()
        start_slab(1, 1)
        plsc.subcore_barrier()

        vr_per_h = n_vr // n_heads
        att_r = [att_v[r, :] for r in range(n_vr)]
        iota = lax.iota(jnp.int32, 16)

        def compute_chunk(b, slot, j):
            def edge_body(e, carry):
                alpha = jnp.zeros((16,), jnp.float32)
                rows = []
                for h in range(n_heads):
                    acc_s = jnp.float32(0.0)
                    for v in range(vr_per_h):
                        r = h * vr_per_h + v
                        sl = pl.ds(r * 16, 16)
                        xjv = xj_v[b, e, sl]
                        xiv = xi_v[b, e, sl]
                        acc_s = acc_s + jnp.sum(
                            _lrelu(xiv + xjv) * att_r[r])
                        rows.append(xjv)
                    alpha = jnp.where(iota == h,
                                      jax.lax.broadcast(acc_s, (16,)), alpha)
                w = jnp.exp(alpha)
                for h in range(n_heads):
                    wh = jax.lax.broadcast(w[h], (16,))
                    for v in range(vr_per_h):
                        r = h * vr_per_h + v
                        ov[b, e, pl.ds(r * 16, 16)] = rows[r] * wh
                ov[b, e, pl.ds(width, 16)] = w
                return carry

            lax.fori_loop(0, CH, edge_body, 0)

        def start_gathers(b, slot, j):
            pltpu.async_copy(tl.at[sidx.at[slot, j]], xj_v.at[b], gj[b])
            pltpu.async_copy(tr.at[didx.at[slot, j]], xi_v.at[b], gi[b])

        def wait_gathers(b, slot, j):
            pltpu.make_async_copy(tl.at[sidx.at[slot, j]], xj_v.at[b],
                                  gj[b]).wait()
            pltpu.make_async_copy(tr.at[didx.at[slot, j]], xi_v.at[b],
                                  gi[b]).wait()

        start_gathers(0, 0, 0)
        start_gathers(1, 0, 1)

        def group_body(g, carry):
            g3 = g % 3

            @pl.when(g + 1 < n_grp)
            def _():
                wait_slab()

            @pl.when(g + 2 < n_grp)
            def _():
                start_slab(g + 2, (g + 2) % 3)

            for j in range(G):
                b = j % 2
                k = g * G + j
                wait_gathers(b, g3, j)

                @pl.when(k >= 2)
                def _():
                    pltpu.make_async_copy(ov.at[b], acc.at[didx.at[g3, j]],
                                          ss[b]).wait()

                compute_chunk(b, g3, j)
                pltpu.async_copy(ov.at[b], acc.at[didx.at[g3, j]], ss[b],
                                 add=True)
                if j < G - 2:
                    start_gathers(b, g3, j + 2)
                else:
                    @pl.when(g + 1 < n_grp)
                    def _():
                        start_gathers(b, (g + 1) % 3, j + 2 - G)
            return carry

        lax.fori_loop(0, n_grp, group_body, 0)
        for b in range(2):
            pltpu.make_async_copy(ov.at[b], acc.at[didx.at[0, 0]],
                                  ss[b]).wait()
        plsc.subcore_barrier()
        pltpu.sync_copy(acc.at[pl.ds(r0, rows_per_tile)],
                        out.at[pl.ds(c * NP + r0, rows_per_tile)])

    return fused_kernel


def _project(x, wl, bl, wr, br):
    n, d = x.shape
    f = wl.shape[1]

    def body(x_ref, wl_ref, bl_ref, wr_ref, br_ref, xl_ref, xr_ref):
        xb = x_ref[...]
        xl_ref[...] = jnp.dot(xb, wl_ref[...],
                              preferred_element_type=jnp.float32) + bl_ref[...]
        xr_ref[...] = jnp.dot(xb, wr_ref[...],
                              preferred_element_type=jnp.float32) + br_ref[...]

    return pl.pallas_call(
        body,
        grid=(n // BN,),
        in_specs=[
            pl.BlockSpec((BN, d), lambda i: (i, 0)),
            pl.BlockSpec((d, f), lambda i: (0, 0)),
            pl.BlockSpec((1, f), lambda i: (0, 0)),
            pl.BlockSpec((d, f), lambda i: (0, 0)),
            pl.BlockSpec((1, f), lambda i: (0, 0)),
        ],
        out_specs=[pl.BlockSpec((BN, f), lambda i: (i, 0))] * 2,
        out_shape=[jax.ShapeDtypeStruct((n, f), jnp.float32)] * 2,
    )(x, wl, bl.reshape(1, -1), wr, br.reshape(1, -1))


def _mid_stage(acc0, acc1, xl1, xr1, a_mat, b_mat, bias1,
               wl2p, bl2p, wr2p, br2p, att2p):
    def body(a0_ref, a1_ref, xl_ref, xr_ref, a_ref, b_ref, b1_ref,
             wl2_ref, bl2_ref, wr2_ref, br2_ref, att2_ref,
             xl2_ref, xr2_ref, s2_ref):
        xl_ = xl_ref[...]
        xr_ = xr_ref[...]
        l = _lrelu(xl_ + xr_)
        wii = jnp.exp(jnp.dot(l, a_ref[...],
                              preferred_element_type=jnp.float32))
        t0 = a0_ref[...]
        t1 = a1_ref[...]
        num = (t0[:, :F1] + t1[:, :F1]
               + jnp.dot(wii, b_ref[...],
                         preferred_element_type=jnp.float32) * xl_)
        den = t0[:, F1:F1 + H1] + t1[:, F1:F1 + H1] + wii
        inv = 1.0 / (den + 1e-16)
        hin = num * jnp.dot(inv, b_ref[...],
                            preferred_element_type=jnp.float32) + b1_ref[...]
        h = jnp.where(hin > 0, hin, jnp.exp(jnp.minimum(hin, 0.0)) - 1.0)
        xl2 = jnp.dot(h, wl2_ref[...],
                      preferred_element_type=jnp.float32) + bl2_ref[...]
        xr2 = jnp.dot(h, wr2_ref[...],
                      preferred_element_type=jnp.float32) + br2_ref[...]
        l2 = _lrelu(xl2 + xr2)
        w2 = jnp.exp(jnp.dot(l2, att2_ref[...],
                             preferred_element_type=jnp.float32))
        z = jnp.zeros((xl2.shape[0], ACC2_W - C2P - 1), jnp.float32)
        xl2_ref[...] = xl2
        xr2_ref[...] = xr2
        s2_ref[...] = jnp.concatenate([w2 * xl2, w2, z], axis=1)

    return pl.pallas_call(
        body,
        grid=(N // BN,),
        in_specs=[
            pl.BlockSpec((BN, ACC1_W), lambda i: (i, 0)),
            pl.BlockSpec((BN, ACC1_W), lambda i: (i, 0)),
            pl.BlockSpec((BN, F1), lambda i: (i, 0)),
            pl.BlockSpec((BN, F1), lambda i: (i, 0)),
            pl.BlockSpec((F1, H1), lambda i: (0, 0)),
            pl.BlockSpec((H1, F1), lambda i: (0, 0)),
            pl.BlockSpec((1, F1), lambda i: (0, 0)),
            pl.BlockSpec((F1, C2P), lambda i: (0, 0)),
            pl.BlockSpec((1, C2P), lambda i: (0, 0)),
            pl.BlockSpec((F1, C2P), lambda i: (0, 0)),
            pl.BlockSpec((1, C2P), lambda i: (0, 0)),
            pl.BlockSpec((C2P, 1), lambda i: (0, 0)),
        ],
        out_specs=[
            pl.BlockSpec((BN, C2P), lambda i: (i, 0)),
            pl.BlockSpec((BN, C2P), lambda i: (i, 0)),
            pl.BlockSpec((BN, ACC2_W), lambda i: (i, 0)),
        ],
        out_shape=[
            jax.ShapeDtypeStruct((N, C2P), jnp.float32),
            jax.ShapeDtypeStruct((N, C2P), jnp.float32),
            jax.ShapeDtypeStruct((N, ACC2_W), jnp.float32),
        ],
    )(acc0, acc1, xl1, xr1, a_mat, b_mat, bias1,
      wl2p, bl2p, wr2p, br2p, att2p)


def _final_stage(acc0, acc1, s2, bias2):
    def body(a0_ref, a1_ref, s2_ref, b2_ref, out_ref):
        t = a0_ref[...] + a1_ref[...] + s2_ref[...]
        den = t[:, C2P:C2P + 1]
        out_ref[...] = t[:, :C2] / (den + 1e-16) + b2_ref[...]

    return pl.pallas_call(
        body,
        grid=(N // BN,),
        in_specs=[
            pl.BlockSpec((BN, ACC2_W), lambda i: (i, 0)),
            pl.BlockSpec((BN, ACC2_W), lambda i: (i, 0)),
            pl.BlockSpec((BN, ACC2_W), lambda i: (i, 0)),
            pl.BlockSpec((1, C2), lambda i: (0, 0)),
        ],
        out_specs=pl.BlockSpec((BN, C2), lambda i: (i, 0)),
        out_shape=jax.ShapeDtypeStruct((N, C2), jnp.float32),
    )(acc0, acc1, s2, bias2.reshape(1, -1))


_edge_pass1 = _make_fused_edge_pass(F1, ACC1_W, H1)
_edge_pass2 = _make_fused_edge_pass(C2P, ACC2_W, 1)


def kernel(x, edge_index, Wl1, bl1, Wr1, br1, att1, bias1,
           Wl2, bl2, Wr2, br2, att2, bias2):
    n_grp = E // (NW * G * CH)
    s3d = edge_index[0].reshape(NW, n_grp, G, CH)
    d3d = edge_index[1].reshape(NW, n_grp, G, CH)

    a1 = (att1[:, :, None] * jnp.eye(H1, dtype=jnp.float32)[:, None, :]
          ).reshape(F1, H1)
    bm = jnp.repeat(jnp.eye(H1, dtype=jnp.float32), C1, axis=1)
    att2p = jnp.zeros((C2P, 1), jnp.float32).at[:C2, 0].set(att2[0])
    att2v = att2p.reshape(3, 16)
    wl2p = jnp.pad(Wl2, ((0, 0), (0, C2P - C2)))
    wr2p = jnp.pad(Wr2, ((0, 0), (0, C2P - C2)))
    bl2p = jnp.pad(bl2, (0, C2P - C2)).reshape(1, -1)
    br2p = jnp.pad(br2, (0, C2P - C2)).reshape(1, -1)
    zeros1 = jnp.zeros((NP // NS, ACC1_W), jnp.float32)
    zeros2 = jnp.zeros((NP // NS, ACC2_W), jnp.float32)

    xl1, xr1 = _project(x, Wl1, bl1, Wr1, br1)
    acc1 = _edge_pass1(xl1, xr1, s3d, d3d, att1, zeros1)
    xl2, xr2, s2 = _mid_stage(acc1[:N], acc1[NP:NP + N], xl1, xr1, a1, bm,
                              bias1.reshape(1, -1), wl2p, bl2p, wr2p, br2p,
                              att2p)
    acc2 = _edge_pass2(xl2, xr2, s3d, d3d, att2v, zeros2)
    return _final_stage(acc2[:N], acc2[NP:NP + N], s2, bias2)


# speedup vs baseline: 69.4976x; 1.4541x over previous
import functools

import jax
import jax.numpy as jnp
from jax import lax
from jax.experimental import pallas as pl
from jax.experimental.pallas import tpu as pltpu
from jax.experimental.pallas import tpu_sc as plsc

N = 10000
E = 320000
D = 128
H1, C1 = 8, 16
F1 = H1 * C1
C2 = 40
C2P = 48
ACC1_W = F1 + 16
ACC2_W = 64

NC, NS = 2, 16
NW = NC * NS
CH = 40
G = 10
NP = 10112
BN = 1000


def _lrelu(v):
    return jnp.where(v >= 0, v, 0.2 * v)


_SC_PARAMS = pltpu.CompilerParams(use_tc_tiling_on_sc=False,
                                  needs_layout_passes=False)


def _make_fused_edge_pass(width, acc_w, n_heads, unroll):
    per_w = E // NW
    n_ch = per_w // CH
    n_grp = n_ch // G
    n_vr = width // 16
    rows_per_tile = NP // NS
    mesh = plsc.VectorSubcoreMesh(core_axis_name="c", subcore_axis_name="s")

    @functools.partial(
        pl.kernel,
        out_type=jax.ShapeDtypeStruct((2 * NP, acc_w), jnp.float32),
        mesh=mesh,
        scratch_types=[
            pltpu.VMEM((3, G, CH), jnp.int32),
            pltpu.VMEM((3, G, CH), jnp.int32),
            pltpu.VMEM((2, CH, width), jnp.float32),
            pltpu.VMEM((2, CH, width), jnp.float32),
            pltpu.VMEM((2, CH, acc_w), jnp.float32),
            pltpu.VMEM((width // 16, 16), jnp.float32),
            pltpu.VMEM_SHARED((NP, acc_w), jnp.float32),
            pltpu.SemaphoreType.DMA,
            pltpu.SemaphoreType.DMA,
            pltpu.SemaphoreType.DMA,
            pltpu.SemaphoreType.DMA,
            pltpu.SemaphoreType.DMA,
            pltpu.SemaphoreType.DMA,
            pltpu.SemaphoreType.DMA,
        ],
        compiler_params=_SC_PARAMS,
    )
    def fused_kernel(tl, tr, s3d, d3d, att, zeros, out,
                     sidx, didx, xj_v, xi_v, ov, att_v, acc,
                     gj0, gj1, gi0, gi1, ss0, ss1, slab_sem):
        gj = (gj0, gj1)
        gi = (gi0, gi1)
        ss = (ss0, ss1)
        c = lax.axis_index("c")
        s = lax.axis_index("s")
        wid = c * NS + s

        pltpu.sync_copy(att, att_v)
        r0 = s * rows_per_tile
        pltpu.sync_copy(zeros, acc.at[pl.ds(r0, rows_per_tile)])

        def start_slab(g, slot):
            pltpu.async_copy(s3d.at[wid, g], sidx.at[slot], slab_sem)
            pltpu.async_copy(d3d.at[wid, g], didx.at[slot], slab_sem)

        def wait_slab():
            pltpu.make_async_copy(s3d.at[wid, 0], sidx.at[0],
                                  slab_sem).wait()
            pltpu.make_async_copy(d3d.at[wid, 0], didx.at[0],
                                  slab_sem).wait()

        start_slab(0, 0)
        wait_slab()
        start_slab(1, 1)
        plsc.subcore_barrier()

        vr_per_h = n_vr // n_heads
        att_r = [att_v[r, :] for r in range(n_vr)]
        iota = lax.iota(jnp.int32, 16)
        onehot = [iota == h for h in range(n_heads)]

        def compute_chunk(b, slot, j):
            @plsc.parallel_loop(0, CH, 1, unroll=unroll)
            def edge_body(e):
                alpha = jnp.zeros((16,), jnp.float32)
                for h in range(n_heads):
                    acc_s = jnp.float32(0.0)
                    for v in range(vr_per_h):
                        r = h * vr_per_h + v
                        sl = pl.ds(r * 16, 16)
                        acc_s = acc_s + jnp.sum(
                            _lrelu(xi_v[b, e, sl] + xj_v[b, e, sl])
                            * att_r[r])
                    alpha = jnp.where(onehot[h],
                                      jax.lax.broadcast(acc_s, (16,)), alpha)
                w = jnp.exp(alpha)
                for h in range(n_heads):
                    wh = jax.lax.broadcast(w[h], (16,))
                    for v in range(vr_per_h):
                        r = h * vr_per_h + v
                        sl = pl.ds(r * 16, 16)
                        ov[b, e, sl] = xj_v[b, e, sl] * wh
                ov[b, e, pl.ds(width, 16)] = w

        def start_gathers(b, slot, j):
            pltpu.async_copy(tl.at[sidx.at[slot, j]], xj_v.at[b], gj[b])
            pltpu.async_copy(tr.at[didx.at[slot, j]], xi_v.at[b], gi[b])

        def wait_gathers(b, slot, j):
            pltpu.make_async_copy(tl.at[sidx.at[slot, j]], xj_v.at[b],
                                  gj[b]).wait()
            pltpu.make_async_copy(tr.at[didx.at[slot, j]], xi_v.at[b],
                                  gi[b]).wait()

        start_gathers(0, 0, 0)
        start_gathers(1, 0, 1)

        def group_body(g, carry):
            g3 = g % 3

            @pl.when(g + 1 < n_grp)
            def _():
                wait_slab()

            @pl.when(g + 2 < n_grp)
            def _():
                start_slab(g + 2, (g + 2) % 3)

            for j in range(G):
                b = j % 2
                k = g * G + j
                wait_gathers(b, g3, j)

                @pl.when(k >= 2)
                def _():
                    pltpu.make_async_copy(ov.at[b], acc.at[didx.at[g3, j]],
                                          ss[b]).wait()

                compute_chunk(b, g3, j)
                pltpu.async_copy(ov.at[b], acc.at[didx.at[g3, j]], ss[b],
                                 add=True)
                if j < G - 2:
                    start_gathers(b, g3, j + 2)
                else:
                    @pl.when(g + 1 < n_grp)
                    def _():
                        start_gathers(b, (g + 1) % 3, j + 2 - G)
            return carry

        lax.fori_loop(0, n_grp, group_body, 0)
        for b in range(2):
            pltpu.make_async_copy(ov.at[b], acc.at[didx.at[0, 0]],
                                  ss[b]).wait()
        plsc.subcore_barrier()
        pltpu.sync_copy(acc.at[pl.ds(r0, rows_per_tile)],
                        out.at[pl.ds(c * NP + r0, rows_per_tile)])

    return fused_kernel


def _project(x, wl, bl, wr, br):
    n, d = x.shape
    f = wl.shape[1]

    def body(x_ref, wl_ref, bl_ref, wr_ref, br_ref, xl_ref, xr_ref):
        xb = x_ref[...]
        xl_ref[...] = jnp.dot(xb, wl_ref[...],
                              preferred_element_type=jnp.float32) + bl_ref[...]
        xr_ref[...] = jnp.dot(xb, wr_ref[...],
                              preferred_element_type=jnp.float32) + br_ref[...]

    return pl.pallas_call(
        body,
        grid=(n // BN,),
        in_specs=[
            pl.BlockSpec((BN, d), lambda i: (i, 0)),
            pl.BlockSpec((d, f), lambda i: (0, 0)),
            pl.BlockSpec((1, f), lambda i: (0, 0)),
            pl.BlockSpec((d, f), lambda i: (0, 0)),
            pl.BlockSpec((1, f), lambda i: (0, 0)),
        ],
        out_specs=[pl.BlockSpec((BN, f), lambda i: (i, 0))] * 2,
        out_shape=[jax.ShapeDtypeStruct((n, f), jnp.float32)] * 2,
    )(x, wl, bl.reshape(1, -1), wr, br.reshape(1, -1))


def _mid_stage(acc0, acc1, xl1, xr1, a_mat, b_mat, bias1,
               wl2p, bl2p, wr2p, br2p, att2p):
    def body(a0_ref, a1_ref, xl_ref, xr_ref, a_ref, b_ref, b1_ref,
             wl2_ref, bl2_ref, wr2_ref, br2_ref, att2_ref,
             xl2_ref, xr2_ref, s2_ref):
        xl_ = xl_ref[...]
        xr_ = xr_ref[...]
        l = _lrelu(xl_ + xr_)
        wii = jnp.exp(jnp.dot(l, a_ref[...],
                              preferred_element_type=jnp.float32))
        t0 = a0_ref[...]
        t1 = a1_ref[...]
        num = (t0[:, :F1] + t1[:, :F1]
               + jnp.dot(wii, b_ref[...],
                         preferred_element_type=jnp.float32) * xl_)
        den = t0[:, F1:F1 + H1] + t1[:, F1:F1 + H1] + wii
        inv = 1.0 / (den + 1e-16)
        hin = num * jnp.dot(inv, b_ref[...],
                            preferred_element_type=jnp.float32) + b1_ref[...]
        h = jnp.where(hin > 0, hin, jnp.exp(jnp.minimum(hin, 0.0)) - 1.0)
        xl2 = jnp.dot(h, wl2_ref[...],
                      preferred_element_type=jnp.float32) + bl2_ref[...]
        xr2 = jnp.dot(h, wr2_ref[...],
                      preferred_element_type=jnp.float32) + br2_ref[...]
        l2 = _lrelu(xl2 + xr2)
        w2 = jnp.exp(jnp.dot(l2, att2_ref[...],
                             preferred_element_type=jnp.float32))
        z = jnp.zeros((xl2.shape[0], ACC2_W - C2P - 1), jnp.float32)
        xl2_ref[...] = xl2
        xr2_ref[...] = xr2
        s2_ref[...] = jnp.concatenate([w2 * xl2, w2, z], axis=1)

    return pl.pallas_call(
        body,
        grid=(N // BN,),
        in_specs=[
            pl.BlockSpec((BN, ACC1_W), lambda i: (i, 0)),
            pl.BlockSpec((BN, ACC1_W), lambda i: (i, 0)),
            pl.BlockSpec((BN, F1), lambda i: (i, 0)),
            pl.BlockSpec((BN, F1), lambda i: (i, 0)),
            pl.BlockSpec((F1, H1), lambda i: (0, 0)),
            pl.BlockSpec((H1, F1), lambda i: (0, 0)),
            pl.BlockSpec((1, F1), lambda i: (0, 0)),
            pl.BlockSpec((F1, C2P), lambda i: (0, 0)),
            pl.BlockSpec((1, C2P), lambda i: (0, 0)),
            pl.BlockSpec((F1, C2P), lambda i: (0, 0)),
            pl.BlockSpec((1, C2P), lambda i: (0, 0)),
            pl.BlockSpec((C2P, 1), lambda i: (0, 0)),
        ],
        out_specs=[
            pl.BlockSpec((BN, C2P), lambda i: (i, 0)),
            pl.BlockSpec((BN, C2P), lambda i: (i, 0)),
            pl.BlockSpec((BN, ACC2_W), lambda i: (i, 0)),
        ],
        out_shape=[
            jax.ShapeDtypeStruct((N, C2P), jnp.float32),
            jax.ShapeDtypeStruct((N, C2P), jnp.float32),
            jax.ShapeDtypeStruct((N, ACC2_W), jnp.float32),
        ],
    )(acc0, acc1, xl1, xr1, a_mat, b_mat, bias1,
      wl2p, bl2p, wr2p, br2p, att2p)


def _final_stage(acc0, acc1, s2, bias2):
    def body(a0_ref, a1_ref, s2_ref, b2_ref, out_ref):
        t = a0_ref[...] + a1_ref[...] + s2_ref[...]
        den = t[:, C2P:C2P + 1]
        out_ref[...] = t[:, :C2] / (den + 1e-16) + b2_ref[...]

    return pl.pallas_call(
        body,
        grid=(N // BN,),
        in_specs=[
            pl.BlockSpec((BN, ACC2_W), lambda i: (i, 0)),
            pl.BlockSpec((BN, ACC2_W), lambda i: (i, 0)),
            pl.BlockSpec((BN, ACC2_W), lambda i: (i, 0)),
            pl.BlockSpec((1, C2), lambda i: (0, 0)),
        ],
        out_specs=pl.BlockSpec((BN, C2), lambda i: (i, 0)),
        out_shape=jax.ShapeDtypeStruct((N, C2), jnp.float32),
    )(acc0, acc1, s2, bias2.reshape(1, -1))


_edge_pass1 = _make_fused_edge_pass(F1, ACC1_W, H1, unroll=2)
_edge_pass2 = _make_fused_edge_pass(C2P, ACC2_W, 1, unroll=4)


def kernel(x, edge_index, Wl1, bl1, Wr1, br1, att1, bias1,
           Wl2, bl2, Wr2, br2, att2, bias2):
    n_grp = E // (NW * G * CH)
    s3d = edge_index[0].reshape(NW, n_grp, G, CH)
    d3d = edge_index[1].reshape(NW, n_grp, G, CH)

    a1 = (att1[:, :, None] * jnp.eye(H1, dtype=jnp.float32)[:, None, :]
          ).reshape(F1, H1)
    bm = jnp.repeat(jnp.eye(H1, dtype=jnp.float32), C1, axis=1)
    att2p = jnp.zeros((C2P, 1), jnp.float32).at[:C2, 0].set(att2[0])
    att2v = att2p.reshape(3, 16)
    wl2p = jnp.pad(Wl2, ((0, 0), (0, C2P - C2)))
    wr2p = jnp.pad(Wr2, ((0, 0), (0, C2P - C2)))
    bl2p = jnp.pad(bl2, (0, C2P - C2)).reshape(1, -1)
    br2p = jnp.pad(br2, (0, C2P - C2)).reshape(1, -1)
    zeros1 = jnp.zeros((NP // NS, ACC1_W), jnp.float32)
    zeros2 = jnp.zeros((NP // NS, ACC2_W), jnp.float32)

    xl1, xr1 = _project(x, Wl1, bl1, Wr1, br1)
    acc1 = _edge_pass1(xl1, xr1, s3d, d3d, att1, zeros1)
    xl2, xr2, s2 = _mid_stage(acc1[:N], acc1[NP:NP + N], xl1, xr1, a1, bm,
                              bias1.reshape(1, -1), wl2p, bl2p, wr2p, br2p,
                              att2p)
    acc2 = _edge_pass2(xl2, xr2, s3d, d3d, att2v, zeros2)
    return _final_stage(acc2[:N], acc2[NP:NP + N], s2, bias2)
